# baseline (speedup 1.0000x reference)
import jax
import jax.numpy as jnp
from jax import lax
from jax.experimental import pallas as pl
from jax.experimental.pallas import tpu as pltpu
from jax.experimental.pallas import tpu_sc as plsc

ROWS = 16384
COLS = 200
TABLE_SIZE = 120
LANES = 16

NUM_CORES = 2
NUM_SUBCORES = 16
NUM_WORKERS = NUM_CORES * NUM_SUBCORES

N_TOTAL = ROWS * COLS
N_PER_WORKER = N_TOTAL // NUM_WORKERS
CHUNK = 25600
NUM_CHUNKS = N_PER_WORKER // CHUNK
UNROLL = 8


def _tec_body(x_hbm, tab_hbm, out_hbm, tab_v, buf_v):
    wid = lax.axis_index("s") * NUM_CORES + lax.axis_index("c")
    pltpu.sync_copy(tab_hbm, tab_v)
    base = wid * N_PER_WORKER

    for chunk in range(NUM_CHUNKS):
        off = base + chunk * CHUNK
        pltpu.sync_copy(x_hbm.at[pl.ds(off, CHUNK)], buf_v)

        @plsc.parallel_loop(0, CHUNK, step=LANES * UNROLL)
        def _(i):
            for j in range(UNROLL):
                sl = pl.ds(i + j * LANES, LANES)
                buf_v[sl] = plsc.load_gather(tab_v, [buf_v[sl]])

        pltpu.sync_copy(buf_v, out_hbm.at[pl.ds(off, CHUNK)])


@jax.jit
def kernel(atomic_nums, conv_tensor):
    mesh = plsc.VectorSubcoreMesh(core_axis_name="c", subcore_axis_name="s")
    run = pl.kernel(
        _tec_body,
        out_type=jax.ShapeDtypeStruct((N_TOTAL,), jnp.int32),
        mesh=mesh,
        scratch_types=[
            pltpu.VMEM((TABLE_SIZE,), jnp.int32),
            pltpu.VMEM((CHUNK,), jnp.int32),
        ],
        compiler_params=pltpu.CompilerParams(needs_layout_passes=False),
    )
    out = run(atomic_nums.reshape(-1), conv_tensor)
    return out.reshape(ROWS, COLS)


# speedup vs baseline: 1.8990x; 1.8990x over previous
import jax
import jax.numpy as jnp
from jax import lax
from jax.experimental import pallas as pl
from jax.experimental.pallas import tpu as pltpu
from jax.experimental.pallas import tpu_sc as plsc

ROWS = 16384
COLS = 200
TABLE_SIZE = 120
LANES = 16

NUM_CORES = 2
NUM_SUBCORES = 16
NUM_WORKERS = NUM_CORES * NUM_SUBCORES
ROWS_PER_WORKER = ROWS // NUM_WORKERS
CHUNK_ROWS = 128
NUM_CHUNKS = ROWS_PER_WORKER // CHUNK_ROWS

_WINDOWS = tuple(range(0, COLS - LANES + 1, LANES)) + (COLS - LANES,)


def _tec_body(x_hbm, tab_hbm, out_hbm, tab_v, buf0, buf1, in_sem, out_sem):
    wid = lax.axis_index("s") * NUM_CORES + lax.axis_index("c")
    pltpu.sync_copy(tab_hbm, tab_v)
    base = wid * ROWS_PER_WORKER
    bufs = (buf0, buf1)

    def copy_in(k):
        src = x_hbm.at[pl.ds(base + k * CHUNK_ROWS, CHUNK_ROWS)]
        return pltpu.make_async_copy(src, bufs[k % 2], in_sem)

    def copy_out(k):
        dst = out_hbm.at[pl.ds(base + k * CHUNK_ROWS, CHUNK_ROWS)]
        return pltpu.make_async_copy(bufs[k % 2], dst, out_sem)

    copy_in(0).start()
    for k in range(NUM_CHUNKS):
        copy_in(k).wait()
        if k >= 2:
            copy_out(k - 2).wait()
        if k + 1 < NUM_CHUNKS:
            copy_in(k + 1).start()
        buf = bufs[k % 2]

        @plsc.parallel_loop(0, CHUNK_ROWS, step=1)
        def _(r):
            idxs = [buf[r, pl.ds(c, LANES)] for c in _WINDOWS]
            for c, idx in zip(_WINDOWS, idxs):
                buf[r, pl.ds(c, LANES)] = plsc.load_gather(tab_v, [idx])

        copy_out(k).start()
    copy_out(NUM_CHUNKS - 2).wait()
    copy_out(NUM_CHUNKS - 1).wait()


@jax.jit
def kernel(atomic_nums, conv_tensor):
    mesh = plsc.VectorSubcoreMesh(core_axis_name="c", subcore_axis_name="s")
    run = pl.kernel(
        _tec_body,
        out_type=jax.ShapeDtypeStruct((ROWS, COLS), jnp.int32),
        mesh=mesh,
        scratch_types=[
            pltpu.VMEM((TABLE_SIZE,), jnp.int32),
            pltpu.VMEM((CHUNK_ROWS, COLS), jnp.int32),
            pltpu.VMEM((CHUNK_ROWS, COLS), jnp.int32),
            pltpu.SemaphoreType.DMA,
            pltpu.SemaphoreType.DMA,
        ],
        compiler_params=pltpu.CompilerParams(needs_layout_passes=False),
    )
    return run(atomic_nums, conv_tensor)
